# baseline (speedup 1.0000x reference)
import jax
import jax.numpy as jnp
from jax import lax
from jax.experimental import pallas as pl
from jax.experimental.pallas import tpu as pltpu
from jax.experimental.pallas import tpu_sc as plsc

BATCH = 1024
S = 7
N = 20
CELLS = BATCH * S * S
NC = 2
NS = 16
NW = NC * NS
UNITS = S * (BATCH // 128)
Sf = 7.0


def _sq(x):
    return x * x


def _sqrt16(x):
    xi = plsc.bitcast(x, jnp.int32)
    yi = jnp.int32(0x5F3759DF) - lax.shift_right_arithmetic(xi, 1)
    y = plsc.bitcast(yi, jnp.float32)
    y = y * (1.5 - 0.5 * x * y * y)
    y = y * (1.5 - 0.5 * x * y * y)
    y = y * (1.5 - 0.5 * x * y * y)
    return jnp.where(x == 0.0, 0.0, x * y)


def _body(pred_hbm, targ_hbm, out_hbm, pad_p, pad_t, acc_v, sem0, sem1):
    wid = lax.axis_index("s") * NC + lax.axis_index("c")
    sems = (sem0, sem1)

    def unit_dma(u, buf):
        i = u // 8
        chb = (u % 8) * 128
        cp_p = pltpu.make_async_copy(
            pred_hbm.at[i, :, :, pl.ds(chb, 128)],
            pad_p.at[buf, :, pl.ds(0, S), :], sems[buf])
        cp_t = pltpu.make_async_copy(
            targ_hbm.at[i, :, :, pl.ds(chb, 128)],
            pad_t.at[buf, :, pl.ds(0, S), :], sems[buf])
        return cp_p, cp_t

    def unit_work(u, buf, acc):
        cp_p, cp_t = unit_dma(u, buf)
        cp_p.wait()
        cp_t.wait()

        def group(q, acc):
            j = q // 8
            b16 = (q % 8) * 16

            def pch(c):
                return pad_p[buf, c, j, pl.ds(b16, 16)]

            def tch(c):
                return pad_t[buf, c, j, pl.ds(b16, 16)]

            p = [pch(c) for c in range(10)]
            t = [tch(c) for c in range(10)]
            t4 = t[4]
            m = jnp.where(t4 > 0.0, 1.0, 0.0)
            l_noobj = jnp.where(t4 == 0.0,
                                _sq(p[4] - t4) + _sq(p[9] - t[9]),
                                0.0)
            l_class = _sq(pch(10) - tch(10))
            for c in range(11, 20):
                l_class = l_class + _sq(pch(c) - tch(c))
            C7 = jnp.float32(1.0 / Sf)
            tx = t[2] * C7
            at0 = 0.5 * t[2]
            at1 = 0.5 * t[3]
            lt_t0 = tx - at0
            lt_t1 = tx - at1
            rb_t0 = tx + at0
            rb_t1 = tx + at1
            area2 = t[2] * t[3]
            px = p[2] * C7
            py = p[7] * C7
            inters = []
            denoms = []
            for b in (0, 1):
                h0 = 0.5 * p[5 * b + 2]
                h1 = 0.5 * p[5 * b + 3]
                w = jnp.maximum(jnp.minimum(px + h0, rb_t0)
                                - jnp.maximum(px - h0, lt_t0), 0.0)
                h = jnp.maximum(jnp.minimum(py + h1, rb_t1)
                                - jnp.maximum(py - h1, lt_t1), 0.0)
                inter = w * h
                area1 = p[5 * b + 2] * p[5 * b + 3]
                inters.append(inter)
                denoms.append(area1 + area2 - inter)
            sel = inters[0] * denoms[1] >= inters[1] * denoms[0]
            max_iou = jnp.where(sel, inters[0], inters[1]) \
                / jnp.where(sel, denoms[0], denoms[1])
            pr = [jnp.where(sel, p[j2], p[5 + j2]) for j2 in range(5)]
            tr = [jnp.where(sel, t[j2], t[5 + j2]) for j2 in range(4)]
            l_xy = _sq(pr[0] - tr[0]) + _sq(pr[1] - tr[1])
            l_wh = pr[2] + tr[2] - 2.0 * _sqrt16(pr[2] * tr[2]) \
                + pr[3] + tr[3] - 2.0 * _sqrt16(pr[3] * tr[3])
            l_obj = _sq(pr[4] - max_iou)
            return acc + (m * (5.0 * (l_xy + l_wh) + l_obj)
                          + l_class * m + 0.5 * l_noobj)

        return lax.fori_loop(0, 7 * 8, group, acc, unroll=2)

    cp0_p, cp0_t = unit_dma(wid, 0)
    cp0_p.start()
    cp0_t.start()
    has2 = wid < UNITS - NW

    @pl.when(has2)
    def _():
        cp1_p, cp1_t = unit_dma(wid + NW, 1)
        cp1_p.start()
        cp1_t.start()

    acc = unit_work(wid, 0, jnp.zeros((16,), jnp.float32))
    acc = lax.cond(has2,
                   lambda a: unit_work(wid + NW, 1, a),
                   lambda a: a,
                   acc)
    acc_v[...] = acc
    pltpu.sync_copy(acc_v, out_hbm.at[wid])


@jax.jit
def _yolo_sc(pred_4d, targ_4d):
    pred_t = jnp.transpose(pred_4d, (1, 3, 2, 0))
    targ_t = jnp.transpose(targ_4d, (1, 3, 2, 0))
    mesh = plsc.VectorSubcoreMesh(
        core_axis_name="c", subcore_axis_name="s",
        num_cores=NC, num_subcores=NS)
    run = pl.kernel(
        _body,
        out_type=jax.ShapeDtypeStruct((NW, 16), jnp.float32),
        mesh=mesh,
        scratch_types=[
            pltpu.VMEM((2, N, 8, 128), jnp.float32),
            pltpu.VMEM((2, N, 8, 128), jnp.float32),
            pltpu.VMEM((16,), jnp.float32),
            pltpu.SemaphoreType.DMA,
            pltpu.SemaphoreType.DMA,
        ],
        compiler_params=pltpu.CompilerParams(needs_layout_passes=False),
    )
    partials = run(pred_t, targ_t)
    return jnp.sum(partials) * (1.0 / BATCH)


def kernel(pred_tensor, target_tensor):
    return _yolo_sc(pred_tensor, target_tensor)


# speedup vs baseline: 1.0356x; 1.0356x over previous
import jax
import jax.numpy as jnp
from jax import lax
from jax.experimental import pallas as pl
from jax.experimental.pallas import tpu as pltpu
from jax.experimental.pallas import tpu_sc as plsc

BATCH = 1024
S = 7
N = 20
CELLS = BATCH * S * S
NC = 2
NS = 16
NW = NC * NS
UNITS = S * (BATCH // 128)
Sf = 7.0


def _sq(x):
    return x * x


def _sqrt16(x):
    xi = plsc.bitcast(x, jnp.int32)
    yi = jnp.int32(0x5F3759DF) - lax.shift_right_arithmetic(xi, 1)
    y = plsc.bitcast(yi, jnp.float32)
    y = y * (1.5 - 0.5 * x * y * y)
    y = y * (1.5 - 0.5 * x * y * y)
    y = y * (1.5 - 0.5 * x * y * y)
    return jnp.where(x == 0.0, 0.0, x * y)


def _body(pred_hbm, targ_hbm, out_hbm, pad_p, pad_t, acc_v, sem0, sem1):
    wid = lax.axis_index("s") * NC + lax.axis_index("c")
    sems = (sem0, sem1)

    def unit_dma(u, buf):
        i = u // 8
        chb = (u % 8) * 128
        cp_p = pltpu.make_async_copy(
            pred_hbm.at[i, :, :, pl.ds(chb, 128)],
            pad_p.at[buf, :, pl.ds(0, S), :], sems[buf])
        cp_t = pltpu.make_async_copy(
            targ_hbm.at[i, :, :, pl.ds(chb, 128)],
            pad_t.at[buf, :, pl.ds(0, S), :], sems[buf])
        return cp_p, cp_t

    def unit_work(u, buf, acc):
        def group(q, acc):
            j = q // 8
            b16 = (q % 8) * 16

            def pch(c):
                return pad_p[buf, c, j, pl.ds(b16, 16)]

            def tch(c):
                return pad_t[buf, c, j, pl.ds(b16, 16)]

            p = [pch(c) for c in range(10)]
            t = [tch(c) for c in range(10)]
            t4 = t[4]
            m = jnp.where(t4 > 0.0, 1.0, 0.0)
            l_noobj = jnp.where(t4 == 0.0,
                                _sq(p[4] - t4) + _sq(p[9] - t[9]),
                                0.0)
            l_class = _sq(pch(10) - tch(10))
            for c in range(11, 20):
                l_class = l_class + _sq(pch(c) - tch(c))
            C7 = jnp.float32(1.0 / Sf)
            tx = t[2] * C7
            at0 = 0.5 * t[2]
            at1 = 0.5 * t[3]
            lt_t0 = tx - at0
            lt_t1 = tx - at1
            rb_t0 = tx + at0
            rb_t1 = tx + at1
            area2 = t[2] * t[3]
            px = p[2] * C7
            py = p[7] * C7
            inters = []
            denoms = []
            for b in (0, 1):
                h0 = 0.5 * p[5 * b + 2]
                h1 = 0.5 * p[5 * b + 3]
                w = jnp.maximum(jnp.minimum(px + h0, rb_t0)
                                - jnp.maximum(px - h0, lt_t0), 0.0)
                h = jnp.maximum(jnp.minimum(py + h1, rb_t1)
                                - jnp.maximum(py - h1, lt_t1), 0.0)
                inter = w * h
                area1 = p[5 * b + 2] * p[5 * b + 3]
                inters.append(inter)
                denoms.append(area1 + area2 - inter)
            sel = inters[0] * denoms[1] >= inters[1] * denoms[0]
            max_iou = jnp.where(sel, inters[0], inters[1]) \
                / jnp.where(sel, denoms[0], denoms[1])
            pr = [jnp.where(sel, p[j2], p[5 + j2]) for j2 in range(5)]
            tr = [jnp.where(sel, t[j2], t[5 + j2]) for j2 in range(4)]
            l_xy = _sq(pr[0] - tr[0]) + _sq(pr[1] - tr[1])
            l_wh = pr[2] + tr[2] - 2.0 * _sqrt16(pr[2] * tr[2]) \
                + pr[3] + tr[3] - 2.0 * _sqrt16(pr[3] * tr[3])
            l_obj = _sq(pr[4] - max_iou)
            return acc + (m * (5.0 * (l_xy + l_wh) + l_obj)
                          + l_class * m + 0.5 * l_noobj)

        return lax.fori_loop(0, 7 * 8, group, acc)

    cp0_p, cp0_t = unit_dma(wid, 0)
    cp0_p.start()
    cp0_t.start()
    has2 = wid < UNITS - NW

    @pl.when(has2)
    def _():
        cp1_p, cp1_t = unit_dma(wid + NW, 1)
        cp1_p.start()
        cp1_t.start()

    def per_unit(k, acc):
        valid = jnp.logical_or(k == 0, has2)

        @pl.when(k == 0)
        def _():
            a, b = unit_dma(wid, 0)
            a.wait()
            b.wait()

        @pl.when(jnp.logical_and(k == 1, has2))
        def _():
            a, b = unit_dma(wid + NW, 1)
            a.wait()
            b.wait()

        return lax.cond(valid,
                        lambda a: unit_work(wid + NW * k, k, a),
                        lambda a: a,
                        acc)

    acc = lax.fori_loop(0, 2, per_unit, jnp.zeros((16,), jnp.float32))
    acc_v[...] = acc
    pltpu.sync_copy(acc_v, out_hbm.at[wid])


@jax.jit
def _yolo_sc(pred_4d, targ_4d):
    pred_t = jnp.transpose(pred_4d, (1, 3, 2, 0))
    targ_t = jnp.transpose(targ_4d, (1, 3, 2, 0))
    mesh = plsc.VectorSubcoreMesh(
        core_axis_name="c", subcore_axis_name="s",
        num_cores=NC, num_subcores=NS)
    run = pl.kernel(
        _body,
        out_type=jax.ShapeDtypeStruct((NW, 16), jnp.float32),
        mesh=mesh,
        scratch_types=[
            pltpu.VMEM((2, N, 8, 128), jnp.float32),
            pltpu.VMEM((2, N, 8, 128), jnp.float32),
            pltpu.VMEM((16,), jnp.float32),
            pltpu.SemaphoreType.DMA,
            pltpu.SemaphoreType.DMA,
        ],
        compiler_params=pltpu.CompilerParams(needs_layout_passes=False),
    )
    partials = run(pred_t, targ_t)
    return jnp.sum(partials) * (1.0 / BATCH)


def kernel(pred_tensor, target_tensor):
    return _yolo_sc(pred_tensor, target_tensor)
